# baseline (speedup 1.0000x reference)
import functools

import jax
import jax.numpy as jnp
from jax import lax
from jax.experimental import pallas as pl
from jax.experimental.pallas import tpu as pltpu
from jax.experimental.pallas import tpu_sc as plsc

N = 10000
E = 320000
H = 128
DE = 4

NC = 2
NS = 16
NW = NC * NS
CHUNK = 128
NCHUNK = 80
EPW = NCHUNK * CHUNK
E_PAD = NW * EPW
NG = NCHUNK // 2
NPAD = 10240
DUMP = NPAD - 1

NHALF = 2
EPWH = EPW // NHALF
EH = E_PAD // NHALF

GCHUNK = 64
GNCHUNK = EPWH // GCHUNK
GNCK0 = 96
GNCK1 = (NHALF * GNCHUNK) - GNCK0
MAXG = max(GNCK0, GNCK1)
GCO1 = NS * GNCK0
NBUF = 4
SNCH = EPWH // CHUNK

F32 = jnp.float32

_sc_mesh = plsc.VectorSubcoreMesh(core_axis_name="c", subcore_axis_name="s")


@functools.partial(
    pl.kernel,
    mesh=_sc_mesh,
    out_type=jax.ShapeDtypeStruct((EH, H), F32),
    scratch_types=(
        [pltpu.VMEM((MAXG, GCHUNK), jnp.int32)] * 2
        + [pltpu.VMEM((GCHUNK, H), F32)] * (2 * NBUF + 2)
        + [pltpu.SemaphoreType.DMA] * (NBUF + 2)
    ),
)
def _sc_gather(a_hbm, b_hbm, ia_hbm, ib_hbm, out_hbm, ia_v, ib_v, *rest):
    row_bufs = rest[:2 * NBUF + 2]
    sems = rest[2 * NBUF + 2:]
    bufs = tuple((row_bufs[2 * b], row_bufs[2 * b + 1], sems[b])
                 for b in range(NBUF))
    sums = tuple((row_bufs[2 * NBUF + p], sems[NBUF + p]) for p in range(2))

    cid = lax.axis_index("c")
    sid = lax.axis_index("s")
    nck = jnp.where(cid == 0, GNCK0, GNCK1)
    cbase = jnp.where(cid == 0, sid * GNCK0, GCO1 + sid * GNCK1)
    ebase = cbase * GCHUNK
    pltpu.sync_copy(ia_hbm.at[pl.ds(cbase, MAXG)], ia_v)
    pltpu.sync_copy(ib_hbm.at[pl.ds(cbase, MAXG)], ib_v)

    def issue_g(j, ba, bb, sem):
        pltpu.async_copy(a_hbm.at[ia_v.at[j]], ba, sem)
        pltpu.async_copy(b_hbm.at[ib_v.at[j]], bb, sem)

    def wait_g(ba, bb, sem):
        pltpu.make_async_copy(a_hbm.at[ia_v.at[0]], ba, sem).wait()
        pltpu.make_async_copy(b_hbm.at[ib_v.at[0]], bb, sem).wait()

    def add_rows(ba, bb, sb):
        def row(i, c):
            for k in range(H // 16):
                sl = pl.ds(k * 16, 16)
                sb[i, sl] = ba[i, sl] + bb[i, sl]
            return c
        lax.fori_loop(0, GCHUNK, row, 0)

    def issue_out(j, sb, sem):
        pltpu.async_copy(sb, out_hbm.at[pl.ds(ebase + j * GCHUNK, GCHUNK)],
                         sem)

    def wait_out(sb, sem):
        pltpu.make_async_copy(sb, out_hbm.at[pl.ds(ebase, GCHUNK)], sem).wait()

    for b in range(NBUF):
        ba, bb, gs = bufs[b]
        issue_g(b, ba, bb, gs)

    def body(g, carry):
        for b in range(NBUF):
            ba, bb, gs = bufs[b]
            sb, osem = sums[b % 2]
            j = g * NBUF + b
            wait_g(ba, bb, gs)

            if b < 2:
                @pl.when(g > 0)
                def _():
                    wait_out(sb, osem)
            else:
                wait_out(sb, osem)

            add_rows(ba, bb, sb)
            issue_out(j, sb, osem)
            jn = jnp.minimum(j + NBUF, nck - 1)
            issue_g(jn, ba, bb, gs)
        return carry

    lax.fori_loop(0, nck // NBUF, body, 0)
    for b in range(NBUF):
        ba, bb, gs = bufs[b]
        wait_g(ba, bb, gs)
    for p in range(2):
        sb, osem = sums[p]
        wait_out(sb, osem)


@functools.partial(
    pl.kernel,
    mesh=_sc_mesh,
    out_type=jax.ShapeDtypeStruct((NC, NPAD, H), F32),
    scratch_types=[
        pltpu.VMEM((SNCH, CHUNK), jnp.int32),
        pltpu.VMEM((CHUNK, H), F32),
        pltpu.VMEM((CHUNK, H), F32),
        pltpu.VMEM_SHARED((NPAD, H), F32),
        pltpu.SemaphoreType.DMA,
        pltpu.SemaphoreType.DMA,
    ],
)
def _sc_scatter(msg_hbm, idx_hbm, zeros_hbm, out_hbm, idx_v, r0, r1, agg_sh,
                m0, m1):
    cid = lax.axis_index("c")
    sid = lax.axis_index("s")
    wid = sid * NC + cid
    base = wid * EPWH
    rows_per_tile = NPAD // NS

    pltpu.sync_copy(zeros_hbm.at[pl.ds(sid * rows_per_tile, rows_per_tile)],
                    agg_sh.at[pl.ds(sid * rows_per_tile, rows_per_tile)])
    pltpu.sync_copy(idx_hbm.at[wid], idx_v)
    plsc.subcore_barrier()

    bufs = ((r0, m0), (r1, m1))

    def issue_r(j, rb, sem):
        pltpu.async_copy(msg_hbm.at[pl.ds(base + j * CHUNK, CHUNK)], rb, sem)

    def wait_r(rb, sem):
        pltpu.make_async_copy(msg_hbm.at[pl.ds(base, CHUNK)], rb, sem).wait()

    issue_r(0, r0, m0)
    issue_r(1, r1, m1)

    def body(g, carry):
        for b in range(2):
            rb, sem = bufs[b]
            j = g * 2 + b
            wait_r(rb, sem)
            pltpu.sync_copy(rb, agg_sh.at[idx_v.at[j]], add=True)
            jn = jnp.minimum(j + 2, SNCH - 1)
            issue_r(jn, rb, sem)
        return carry

    lax.fori_loop(0, SNCH // 2, body, 0)
    for b in range(2):
        rb, sem = bufs[b]
        wait_r(rb, sem)
    plsc.subcore_barrier()

    pltpu.sync_copy(agg_sh.at[pl.ds(sid * rows_per_tile, rows_per_tile)],
                    out_hbm.at[cid, pl.ds(sid * rows_per_tile, rows_per_tile)])


NB = 1024
MB = 2048


def _dot(a, b):
    return jnp.dot(a, b, preferred_element_type=F32)


def _embed_body(x_ref, w_ref, b_ref, wa_ref, wb_ref, h_ref, a_ref, bt_ref):
    h = _dot(x_ref[...], w_ref[...]) + b_ref[...]
    h_ref[...] = h
    a_ref[...] = _dot(h, wa_ref[...])
    bt_ref[...] = _dot(h, wb_ref[...])


def _msg_body(pre_ref, ea_ref, wc_ref, b1_ref, w2_ref, b2_ref, out_ref):
    acc = pre_ref[...] + b1_ref[...]
    ea = ea_ref[...]
    for k in range(DE):
        acc = acc + ea[:, k:k + 1] * wc_ref[k:k + 1, :]
    m = jnp.maximum(acc, 0.0)
    m = jnp.maximum(_dot(m, w2_ref[...]) + b2_ref[...], 0.0)
    out_ref[...] = m


NAGG = NC * NHALF


def _sum_aggs(g_refs):
    s = g_refs[0][...]
    for g in g_refs[1:]:
        s = s + g[...]
    return s


def _update_body(h_ref, *rest):
    g_refs = rest[:NAGG]
    (u1h_ref, u1a_ref, u1b_ref, u2w_ref, u2b_ref, wa_ref, wb_ref,
     h_out, a_out, b_out) = rest[NAGG:]
    hv = h_ref[...]
    agg = _sum_aggs(g_refs)
    u = jnp.maximum(_dot(hv, u1h_ref[...]) + _dot(agg, u1a_ref[...])
                    + u1b_ref[...], 0.0)
    u = _dot(u, u2w_ref[...]) + u2b_ref[...]
    hn = hv + u
    h_out[...] = hn
    a_out[...] = _dot(hn, wa_ref[...])
    b_out[...] = _dot(hn, wb_ref[...])


def _final_body(h_ref, *rest):
    g_refs = rest[:NAGG]
    (u1h_ref, u1a_ref, u1b_ref, u2w_ref, u2b_ref, pp1_ref, pp1b_ref,
     pp2_ref, pp2b_ref, po1_ref, po1b_ref, po2_ref, po2b_ref,
     out_ref) = rest[NAGG:]
    hv = h_ref[...]
    agg = _sum_aggs(g_refs)
    u = jnp.maximum(_dot(hv, u1h_ref[...]) + _dot(agg, u1a_ref[...])
                    + u1b_ref[...], 0.0)
    u = _dot(u, u2w_ref[...]) + u2b_ref[...]
    hn = hv + u
    pp = jnp.maximum(_dot(hn, pp1_ref[...]) + pp1b_ref[...], 0.0)
    pp = _dot(pp, pp2_ref[...]) + pp2b_ref[...]
    q = jnp.maximum(_dot(pp, po1_ref[...]) + po1b_ref[...], 0.0)
    out_ref[...] = _dot(q, po2_ref[...]) + po2b_ref[...]


def _node_spec():
    return pl.BlockSpec((NB, H), lambda i: (i, 0))


def _full_spec(shape):
    return pl.BlockSpec(shape, lambda i: tuple(0 for _ in shape))


def _edge_spec(cols):
    return pl.BlockSpec((MB, cols), lambda i: (i, 0))


_embed_call = pl.pallas_call(
    _embed_body,
    grid=(NPAD // NB,),
    in_specs=[_node_spec(), _full_spec((H, H)), _full_spec((1, H)),
              _full_spec((H, H)), _full_spec((H, H))],
    out_specs=[_node_spec(), _node_spec(), _node_spec()],
    out_shape=[jax.ShapeDtypeStruct((NPAD, H), F32)] * 3,
)

_msg_call = pl.pallas_call(
    _msg_body,
    grid=(EH // MB,),
    in_specs=[_edge_spec(H), _edge_spec(DE),
              _full_spec((DE, H)), _full_spec((1, H)), _full_spec((H, H)),
              _full_spec((1, H))],
    out_specs=_edge_spec(H),
    out_shape=jax.ShapeDtypeStruct((EH, H), F32),
)

_update_call = pl.pallas_call(
    _update_body,
    grid=(NPAD // NB,),
    in_specs=[_node_spec()] * (1 + NAGG)
             + [_full_spec((H, H)), _full_spec((H, H)), _full_spec((1, H)),
                _full_spec((H, H)), _full_spec((1, H)),
                _full_spec((H, H)), _full_spec((H, H))],
    out_specs=[_node_spec(), _node_spec(), _node_spec()],
    out_shape=[jax.ShapeDtypeStruct((NPAD, H), F32)] * 3,
)

_final_call = pl.pallas_call(
    _final_body,
    grid=(NPAD // NB,),
    in_specs=[_node_spec()] * (1 + NAGG)
             + [_full_spec((H, H)), _full_spec((H, H)), _full_spec((1, H)),
                _full_spec((H, H)), _full_spec((1, H)),
                _full_spec((H, H)), _full_spec((1, H)),
                _full_spec((H, H)), _full_spec((1, H)),
                _full_spec((H, H)), _full_spec((1, H)),
                _full_spec((H, H)), _full_spec((1, H))],
    out_specs=_node_spec(),
    out_shape=jax.ShapeDtypeStruct((NPAD, H), F32),
)


def _row(v):
    return v.reshape(1, H)


def kernel(x, edge_index, edge_attr, params):
    ei = edge_index.astype(jnp.int32)
    src, dst = ei[0], ei[1]
    pad = E_PAD - E
    dst_p = jnp.concatenate([dst, jnp.zeros((pad,), jnp.int32)]) \
        .reshape(NW, NHALF, EPWH)
    src_p = jnp.concatenate([src, jnp.zeros((pad,), jnp.int32)]) \
        .reshape(NW, NHALF, EPWH)
    dst_s = jnp.concatenate([dst, jnp.full((pad,), DUMP, jnp.int32)]) \
        .reshape(NW, NHALF, EPWH)
    ea_p = jnp.concatenate([edge_attr.astype(F32),
                            jnp.zeros((pad, DE), F32)], axis=0) \
        .reshape(NW, NHALF, EPWH, DE)
    zpad = jnp.zeros((MAXG, GCHUNK), jnp.int32)
    idx_a = [jnp.concatenate(
        [dst_p[:, hf].reshape(EH // GCHUNK, GCHUNK), zpad])
        for hf in range(NHALF)]
    idx_b = [jnp.concatenate(
        [src_p[:, hf].reshape(EH // GCHUNK, GCHUNK), zpad])
        for hf in range(NHALF)]
    idx_s = [dst_s[:, hf].reshape(NW, SNCH, CHUNK) for hf in range(NHALF)]
    ea_h = [ea_p[:, hf].reshape(EH, DE) for hf in range(NHALF)]
    x_p = jnp.concatenate([x.astype(F32), jnp.zeros((NPAD - N, H), F32)],
                          axis=0)
    zeros_npad = jnp.zeros((NPAD, H), F32)

    layers = params['layers']
    wa0 = layers[0]['m1W'][:H]
    wb0 = layers[0]['m1W'][H:2 * H]
    h, A, B = _embed_call(x_p, params['emb_W'], _row(params['emb_b']),
                          wa0, wb0)

    out = None
    for li in range(len(layers)):
        lp = layers[li]
        wc = lp['m1W'][2 * H:]
        aggs = []
        pres = [None] * NHALF
        msgs = [None] * NHALF
        for hf in range(NHALF):
            pres[hf] = _sc_gather(A, B, idx_a[hf], idx_b[hf])
        for hf in range(NHALF):
            msgs[hf] = _msg_call(pres[hf], ea_h[hf], wc, _row(lp['m1b']),
                                 lp['m2W'], _row(lp['m2b']))
        for hf in range(NHALF):
            ap = _sc_scatter(msgs[hf], idx_s[hf], zeros_npad)
            aggs.extend([ap[0], ap[1]])
        u1h = lp['u1W'][:H]
        u1a = lp['u1W'][H:]
        if li + 1 < len(layers):
            nxt = layers[li + 1]
            h, A, B = _update_call(h, *aggs,
                                   u1h, u1a, _row(lp['u1b']), lp['u2W'],
                                   _row(lp['u2b']),
                                   nxt['m1W'][:H], nxt['m1W'][H:2 * H])
        else:
            out = _final_call(h, *aggs,
                              u1h, u1a, _row(lp['u1b']), lp['u2W'],
                              _row(lp['u2b']),
                              params['pp1W'], _row(params['pp1b']),
                              params['pp2W'], _row(params['pp2b']),
                              params['po1W'], _row(params['po1b']),
                              params['po2W'], _row(params['po2b']))
    return out[:N]


# speedup vs baseline: 1.0302x; 1.0302x over previous
import functools

import jax
import jax.numpy as jnp
from jax import lax
from jax.experimental import pallas as pl
from jax.experimental.pallas import tpu as pltpu
from jax.experimental.pallas import tpu_sc as plsc

N = 10000
E = 320000
H = 128
DE = 4

NC = 2
NS = 16
NW = NC * NS
CHUNK = 128
NCHUNK = 80
EPW = NCHUNK * CHUNK
E_PAD = NW * EPW
NG = NCHUNK // 2
NPAD = 10240
DUMP = NPAD - 1

NHALF = 2
EPWH = EPW // NHALF
EH = E_PAD // NHALF

GCHUNK = 64
GNCHUNK = EPWH // GCHUNK
GNCK0 = 136
GNCK1 = (NHALF * GNCHUNK) - GNCK0
MAXG = max(GNCK0, GNCK1)
GCO1 = NS * GNCK0
NBUF = 4
SNCH = EPWH // CHUNK

F32 = jnp.float32

_sc_mesh = plsc.VectorSubcoreMesh(core_axis_name="c", subcore_axis_name="s")


@functools.partial(
    pl.kernel,
    mesh=_sc_mesh,
    out_type=jax.ShapeDtypeStruct((EH, H), F32),
    scratch_types=(
        [pltpu.VMEM((MAXG, GCHUNK), jnp.int32)] * 2
        + [pltpu.VMEM((GCHUNK, H), F32)] * (2 * NBUF + 2)
        + [pltpu.SemaphoreType.DMA] * (NBUF + 2)
    ),
)
def _sc_gather(a_hbm, b_hbm, ia_hbm, ib_hbm, out_hbm, ia_v, ib_v, *rest):
    row_bufs = rest[:2 * NBUF + 2]
    sems = rest[2 * NBUF + 2:]
    bufs = tuple((row_bufs[2 * b], row_bufs[2 * b + 1], sems[b])
                 for b in range(NBUF))
    sums = tuple((row_bufs[2 * NBUF + p], sems[NBUF + p]) for p in range(2))

    cid = lax.axis_index("c")
    sid = lax.axis_index("s")
    nck = jnp.where(cid == 0, GNCK0, GNCK1)
    cbase = jnp.where(cid == 0, sid * GNCK0, GCO1 + sid * GNCK1)
    ebase = cbase * GCHUNK
    pltpu.sync_copy(ia_hbm.at[pl.ds(cbase, MAXG)], ia_v)
    pltpu.sync_copy(ib_hbm.at[pl.ds(cbase, MAXG)], ib_v)

    def issue_g(j, ba, bb, sem):
        pltpu.async_copy(a_hbm.at[ia_v.at[j]], ba, sem)
        pltpu.async_copy(b_hbm.at[ib_v.at[j]], bb, sem)

    def wait_g(ba, bb, sem):
        pltpu.make_async_copy(a_hbm.at[ia_v.at[0]], ba, sem).wait()
        pltpu.make_async_copy(b_hbm.at[ib_v.at[0]], bb, sem).wait()

    def add_rows(ba, bb, sb):
        def row(i, c):
            for k in range(H // 16):
                sl = pl.ds(k * 16, 16)
                sb[i, sl] = ba[i, sl] + bb[i, sl]
            return c
        lax.fori_loop(0, GCHUNK, row, 0)

    def issue_out(j, sb, sem):
        pltpu.async_copy(sb, out_hbm.at[pl.ds(ebase + j * GCHUNK, GCHUNK)],
                         sem)

    def wait_out(sb, sem):
        pltpu.make_async_copy(sb, out_hbm.at[pl.ds(ebase, GCHUNK)], sem).wait()

    for b in range(NBUF):
        ba, bb, gs = bufs[b]
        issue_g(b, ba, bb, gs)

    def body(g, carry):
        for b in range(NBUF):
            ba, bb, gs = bufs[b]
            sb, osem = sums[b % 2]
            j = g * NBUF + b
            wait_g(ba, bb, gs)

            if b < 2:
                @pl.when(g > 0)
                def _():
                    wait_out(sb, osem)
            else:
                wait_out(sb, osem)

            add_rows(ba, bb, sb)
            issue_out(j, sb, osem)
            jn = jnp.minimum(j + NBUF, nck - 1)
            issue_g(jn, ba, bb, gs)
        return carry

    lax.fori_loop(0, nck // NBUF, body, 0)
    for b in range(NBUF):
        ba, bb, gs = bufs[b]
        wait_g(ba, bb, gs)
    for p in range(2):
        sb, osem = sums[p]
        wait_out(sb, osem)


@functools.partial(
    pl.kernel,
    mesh=_sc_mesh,
    out_type=jax.ShapeDtypeStruct((NC, NPAD, H), F32),
    scratch_types=[
        pltpu.VMEM((SNCH, CHUNK), jnp.int32),
        pltpu.VMEM((CHUNK, H), F32),
        pltpu.VMEM((CHUNK, H), F32),
        pltpu.VMEM_SHARED((NPAD, H), F32),
        pltpu.SemaphoreType.DMA,
        pltpu.SemaphoreType.DMA,
    ],
)
def _sc_scatter(msg_hbm, idx_hbm, zeros_hbm, out_hbm, idx_v, r0, r1, agg_sh,
                m0, m1):
    cid = lax.axis_index("c")
    sid = lax.axis_index("s")
    wid = sid * NC + cid
    base = wid * EPWH
    rows_per_tile = NPAD // NS

    pltpu.sync_copy(zeros_hbm.at[pl.ds(sid * rows_per_tile, rows_per_tile)],
                    agg_sh.at[pl.ds(sid * rows_per_tile, rows_per_tile)])
    pltpu.sync_copy(idx_hbm.at[wid], idx_v)
    plsc.subcore_barrier()

    bufs = ((r0, m0), (r1, m1))

    def issue_r(j, rb, sem):
        pltpu.async_copy(msg_hbm.at[pl.ds(base + j * CHUNK, CHUNK)], rb, sem)

    def wait_r(rb, sem):
        pltpu.make_async_copy(msg_hbm.at[pl.ds(base, CHUNK)], rb, sem).wait()

    issue_r(0, r0, m0)
    issue_r(1, r1, m1)

    def body(g, carry):
        for b in range(2):
            rb, sem = bufs[b]
            j = g * 2 + b
            wait_r(rb, sem)
            pltpu.sync_copy(rb, agg_sh.at[idx_v.at[j]], add=True)
            jn = jnp.minimum(j + 2, SNCH - 1)
            issue_r(jn, rb, sem)
        return carry

    lax.fori_loop(0, SNCH // 2, body, 0)
    for b in range(2):
        rb, sem = bufs[b]
        wait_r(rb, sem)
    plsc.subcore_barrier()

    pltpu.sync_copy(agg_sh.at[pl.ds(sid * rows_per_tile, rows_per_tile)],
                    out_hbm.at[cid, pl.ds(sid * rows_per_tile, rows_per_tile)])


NB = 1024
MB = 2048


def _dot(a, b):
    return jnp.dot(a, b, preferred_element_type=F32)


def _embed_body(x_ref, w_ref, b_ref, wa_ref, wb_ref, h_ref, a_ref, bt_ref):
    h = _dot(x_ref[...], w_ref[...]) + b_ref[...]
    h_ref[...] = h
    a_ref[...] = _dot(h, wa_ref[...])
    bt_ref[...] = _dot(h, wb_ref[...])


def _msg_body(pre_ref, ea_ref, wc_ref, b1_ref, w2_ref, b2_ref, out_ref):
    acc = pre_ref[...] + b1_ref[...]
    ea = ea_ref[...]
    for k in range(DE):
        acc = acc + ea[:, k:k + 1] * wc_ref[k:k + 1, :]
    m = jnp.maximum(acc, 0.0)
    m = jnp.maximum(_dot(m, w2_ref[...]) + b2_ref[...], 0.0)
    out_ref[...] = m


NAGG = NC * NHALF


def _sum_aggs(g_refs):
    s = g_refs[0][...]
    for g in g_refs[1:]:
        s = s + g[...]
    return s


def _update_body(h_ref, *rest):
    g_refs = rest[:NAGG]
    (u1h_ref, u1a_ref, u1b_ref, u2w_ref, u2b_ref, wa_ref, wb_ref,
     h_out, a_out, b_out) = rest[NAGG:]
    hv = h_ref[...]
    agg = _sum_aggs(g_refs)
    u = jnp.maximum(_dot(hv, u1h_ref[...]) + _dot(agg, u1a_ref[...])
                    + u1b_ref[...], 0.0)
    u = _dot(u, u2w_ref[...]) + u2b_ref[...]
    hn = hv + u
    h_out[...] = hn
    a_out[...] = _dot(hn, wa_ref[...])
    b_out[...] = _dot(hn, wb_ref[...])


def _final_body(h_ref, *rest):
    g_refs = rest[:NAGG]
    (u1h_ref, u1a_ref, u1b_ref, u2w_ref, u2b_ref, pp1_ref, pp1b_ref,
     pp2_ref, pp2b_ref, po1_ref, po1b_ref, po2_ref, po2b_ref,
     out_ref) = rest[NAGG:]
    hv = h_ref[...]
    agg = _sum_aggs(g_refs)
    u = jnp.maximum(_dot(hv, u1h_ref[...]) + _dot(agg, u1a_ref[...])
                    + u1b_ref[...], 0.0)
    u = _dot(u, u2w_ref[...]) + u2b_ref[...]
    hn = hv + u
    pp = jnp.maximum(_dot(hn, pp1_ref[...]) + pp1b_ref[...], 0.0)
    pp = _dot(pp, pp2_ref[...]) + pp2b_ref[...]
    q = jnp.maximum(_dot(pp, po1_ref[...]) + po1b_ref[...], 0.0)
    out_ref[...] = _dot(q, po2_ref[...]) + po2b_ref[...]


def _node_spec():
    return pl.BlockSpec((NB, H), lambda i: (i, 0))


def _full_spec(shape):
    return pl.BlockSpec(shape, lambda i: tuple(0 for _ in shape))


def _edge_spec(cols):
    return pl.BlockSpec((MB, cols), lambda i: (i, 0))


_embed_call = pl.pallas_call(
    _embed_body,
    grid=(NPAD // NB,),
    in_specs=[_node_spec(), _full_spec((H, H)), _full_spec((1, H)),
              _full_spec((H, H)), _full_spec((H, H))],
    out_specs=[_node_spec(), _node_spec(), _node_spec()],
    out_shape=[jax.ShapeDtypeStruct((NPAD, H), F32)] * 3,
)

_msg_call = pl.pallas_call(
    _msg_body,
    grid=(EH // MB,),
    in_specs=[_edge_spec(H), _edge_spec(DE),
              _full_spec((DE, H)), _full_spec((1, H)), _full_spec((H, H)),
              _full_spec((1, H))],
    out_specs=_edge_spec(H),
    out_shape=jax.ShapeDtypeStruct((EH, H), F32),
)

_update_call = pl.pallas_call(
    _update_body,
    grid=(NPAD // NB,),
    in_specs=[_node_spec()] * (1 + NAGG)
             + [_full_spec((H, H)), _full_spec((H, H)), _full_spec((1, H)),
                _full_spec((H, H)), _full_spec((1, H)),
                _full_spec((H, H)), _full_spec((H, H))],
    out_specs=[_node_spec(), _node_spec(), _node_spec()],
    out_shape=[jax.ShapeDtypeStruct((NPAD, H), F32)] * 3,
)

_final_call = pl.pallas_call(
    _final_body,
    grid=(NPAD // NB,),
    in_specs=[_node_spec()] * (1 + NAGG)
             + [_full_spec((H, H)), _full_spec((H, H)), _full_spec((1, H)),
                _full_spec((H, H)), _full_spec((1, H)),
                _full_spec((H, H)), _full_spec((1, H)),
                _full_spec((H, H)), _full_spec((1, H)),
                _full_spec((H, H)), _full_spec((1, H)),
                _full_spec((H, H)), _full_spec((1, H))],
    out_specs=_node_spec(),
    out_shape=jax.ShapeDtypeStruct((NPAD, H), F32),
)


def _row(v):
    return v.reshape(1, H)


def kernel(x, edge_index, edge_attr, params):
    ei = edge_index.astype(jnp.int32)
    src, dst = ei[0], ei[1]
    pad = E_PAD - E
    dst_p = jnp.concatenate([dst, jnp.zeros((pad,), jnp.int32)]) \
        .reshape(NW, NHALF, EPWH)
    src_p = jnp.concatenate([src, jnp.zeros((pad,), jnp.int32)]) \
        .reshape(NW, NHALF, EPWH)
    dst_s = jnp.concatenate([dst, jnp.full((pad,), DUMP, jnp.int32)]) \
        .reshape(NW, NHALF, EPWH)
    ea_p = jnp.concatenate([edge_attr.astype(F32),
                            jnp.zeros((pad, DE), F32)], axis=0) \
        .reshape(NW, NHALF, EPWH, DE)
    zpad = jnp.zeros((MAXG, GCHUNK), jnp.int32)
    idx_a = [jnp.concatenate(
        [dst_p[:, hf].reshape(EH // GCHUNK, GCHUNK), zpad])
        for hf in range(NHALF)]
    idx_b = [jnp.concatenate(
        [src_p[:, hf].reshape(EH // GCHUNK, GCHUNK), zpad])
        for hf in range(NHALF)]
    idx_s = [dst_s[:, hf].reshape(NW, SNCH, CHUNK) for hf in range(NHALF)]
    ea_h = [ea_p[:, hf].reshape(EH, DE) for hf in range(NHALF)]
    x_p = jnp.concatenate([x.astype(F32), jnp.zeros((NPAD - N, H), F32)],
                          axis=0)
    zeros_npad = jnp.zeros((NPAD, H), F32)

    layers = params['layers']
    wa0 = layers[0]['m1W'][:H]
    wb0 = layers[0]['m1W'][H:2 * H]
    h, A, B = _embed_call(x_p, params['emb_W'], _row(params['emb_b']),
                          wa0, wb0)

    out = None
    for li in range(len(layers)):
        lp = layers[li]
        wc = lp['m1W'][2 * H:]
        aggs = []
        pres = [None] * NHALF
        msgs = [None] * NHALF
        for hf in range(NHALF):
            pres[hf] = _sc_gather(A, B, idx_a[hf], idx_b[hf])
        for hf in range(NHALF):
            msgs[hf] = _msg_call(pres[hf], ea_h[hf], wc, _row(lp['m1b']),
                                 lp['m2W'], _row(lp['m2b']))
        for hf in range(NHALF):
            ap = _sc_scatter(msgs[hf], idx_s[hf], zeros_npad)
            aggs.extend([ap[0], ap[1]])
        u1h = lp['u1W'][:H]
        u1a = lp['u1W'][H:]
        if li + 1 < len(layers):
            nxt = layers[li + 1]
            h, A, B = _update_call(h, *aggs,
                                   u1h, u1a, _row(lp['u1b']), lp['u2W'],
                                   _row(lp['u2b']),
                                   nxt['m1W'][:H], nxt['m1W'][H:2 * H])
        else:
            out = _final_call(h, *aggs,
                              u1h, u1a, _row(lp['u1b']), lp['u2W'],
                              _row(lp['u2b']),
                              params['pp1W'], _row(params['pp1b']),
                              params['pp2W'], _row(params['pp2b']),
                              params['po1W'], _row(params['po1b']),
                              params['po2W'], _row(params['po2b']))
    return out[:N]


# speedup vs baseline: 1.0761x; 1.0445x over previous
import functools

import jax
import jax.numpy as jnp
from jax import lax
from jax.experimental import pallas as pl
from jax.experimental.pallas import tpu as pltpu
from jax.experimental.pallas import tpu_sc as plsc

N = 10000
E = 320000
H = 128
DE = 4

NC = 2
NS = 16
NW = NC * NS
CHUNK = 128
NCHUNK = 80
EPW = NCHUNK * CHUNK
E_PAD = NW * EPW
NG = NCHUNK // 2
NPAD = 10240
DUMP = NPAD - 1

NHALF = 2
EPWH = EPW // NHALF
EH = E_PAD // NHALF

GCHUNK = 64
GNCHUNK = EPWH // GCHUNK
GNCK0 = 120
GNCK1 = (NHALF * GNCHUNK) - GNCK0
MAXG = max(GNCK0, GNCK1)
GCO1 = NS * GNCK0
NBUF = 4
SNCH = EPWH // CHUNK

F32 = jnp.float32

_sc_mesh = plsc.VectorSubcoreMesh(core_axis_name="c", subcore_axis_name="s")


@functools.partial(
    pl.kernel,
    mesh=_sc_mesh,
    out_type=jax.ShapeDtypeStruct((EH, H), F32),
    scratch_types=(
        [pltpu.VMEM((MAXG, GCHUNK), jnp.int32)] * 2
        + [pltpu.VMEM((GCHUNK, H), F32)] * (2 * NBUF + 2)
        + [pltpu.SemaphoreType.DMA] * (NBUF + 2)
    ),
)
def _sc_gather(a_hbm, b_hbm, ia_hbm, ib_hbm, out_hbm, ia_v, ib_v, *rest):
    row_bufs = rest[:2 * NBUF + 2]
    sems = rest[2 * NBUF + 2:]
    bufs = tuple((row_bufs[2 * b], row_bufs[2 * b + 1], sems[b])
                 for b in range(NBUF))
    sums = tuple((row_bufs[2 * NBUF + p], sems[NBUF + p]) for p in range(2))

    cid = lax.axis_index("c")
    sid = lax.axis_index("s")
    nck = jnp.where(cid == 0, GNCK0, GNCK1)
    cbase = jnp.where(cid == 0, sid * GNCK0, GCO1 + sid * GNCK1)
    ebase = cbase * GCHUNK
    pltpu.sync_copy(ia_hbm.at[pl.ds(cbase, MAXG)], ia_v)
    pltpu.sync_copy(ib_hbm.at[pl.ds(cbase, MAXG)], ib_v)

    def issue_g(j, ba, bb, sem):
        pltpu.async_copy(a_hbm.at[ia_v.at[j]], ba, sem)
        pltpu.async_copy(b_hbm.at[ib_v.at[j]], bb, sem)

    def wait_g(ba, bb, sem):
        pltpu.make_async_copy(a_hbm.at[ia_v.at[0]], ba, sem).wait()
        pltpu.make_async_copy(b_hbm.at[ib_v.at[0]], bb, sem).wait()

    def add_rows(ba, bb, sb):
        def row(i, c):
            for k in range(H // 16):
                sl = pl.ds(k * 16, 16)
                sb[i, sl] = ba[i, sl] + bb[i, sl]
            return c
        lax.fori_loop(0, GCHUNK, row, 0)

    def issue_out(j, sb, sem):
        pltpu.async_copy(sb, out_hbm.at[pl.ds(ebase + j * GCHUNK, GCHUNK)],
                         sem)

    def wait_out(sb, sem):
        pltpu.make_async_copy(sb, out_hbm.at[pl.ds(ebase, GCHUNK)], sem).wait()

    for b in range(NBUF):
        ba, bb, gs = bufs[b]
        issue_g(b, ba, bb, gs)

    def body(g, carry):
        for b in range(NBUF):
            ba, bb, gs = bufs[b]
            sb, osem = sums[b % 2]
            j = g * NBUF + b
            wait_g(ba, bb, gs)

            if b < 2:
                @pl.when(g > 0)
                def _():
                    wait_out(sb, osem)
            else:
                wait_out(sb, osem)

            add_rows(ba, bb, sb)
            issue_out(j, sb, osem)
            jn = jnp.minimum(j + NBUF, nck - 1)
            issue_g(jn, ba, bb, gs)
        return carry

    lax.fori_loop(0, nck // NBUF, body, 0)
    for b in range(NBUF):
        ba, bb, gs = bufs[b]
        wait_g(ba, bb, gs)
    for p in range(2):
        sb, osem = sums[p]
        wait_out(sb, osem)


@functools.partial(
    pl.kernel,
    mesh=_sc_mesh,
    out_type=jax.ShapeDtypeStruct((NC, NPAD, H), F32),
    scratch_types=[
        pltpu.VMEM((SNCH, CHUNK), jnp.int32),
        pltpu.VMEM((CHUNK, H), F32),
        pltpu.VMEM((CHUNK, H), F32),
        pltpu.VMEM_SHARED((NPAD, H), F32),
        pltpu.SemaphoreType.DMA,
        pltpu.SemaphoreType.DMA,
    ],
)
def _sc_scatter(msg_hbm, idx_hbm, zeros_hbm, out_hbm, idx_v, r0, r1, agg_sh,
                m0, m1):
    cid = lax.axis_index("c")
    sid = lax.axis_index("s")
    wid = sid * NC + cid
    base = wid * EPWH
    rows_per_tile = NPAD // NS

    pltpu.sync_copy(zeros_hbm.at[pl.ds(sid * rows_per_tile, rows_per_tile)],
                    agg_sh.at[pl.ds(sid * rows_per_tile, rows_per_tile)])
    pltpu.sync_copy(idx_hbm.at[wid], idx_v)
    plsc.subcore_barrier()

    bufs = ((r0, m0), (r1, m1))

    def issue_r(j, rb, sem):
        pltpu.async_copy(msg_hbm.at[pl.ds(base + j * CHUNK, CHUNK)], rb, sem)

    def wait_r(rb, sem):
        pltpu.make_async_copy(msg_hbm.at[pl.ds(base, CHUNK)], rb, sem).wait()

    issue_r(0, r0, m0)
    issue_r(1, r1, m1)

    def body(g, carry):
        for b in range(2):
            rb, sem = bufs[b]
            j = g * 2 + b
            wait_r(rb, sem)
            pltpu.sync_copy(rb, agg_sh.at[idx_v.at[j]], add=True)
            jn = jnp.minimum(j + 2, SNCH - 1)
            issue_r(jn, rb, sem)
        return carry

    lax.fori_loop(0, SNCH // 2, body, 0)
    for b in range(2):
        rb, sem = bufs[b]
        wait_r(rb, sem)
    plsc.subcore_barrier()

    pltpu.sync_copy(agg_sh.at[pl.ds(sid * rows_per_tile, rows_per_tile)],
                    out_hbm.at[cid, pl.ds(sid * rows_per_tile, rows_per_tile)])


NB = 1024
MB = 2048


def _dot(a, b):
    return jnp.dot(a, b, preferred_element_type=F32)


def _embed_body(x_ref, w_ref, b_ref, wa_ref, wb_ref, h_ref, a_ref, bt_ref):
    h = _dot(x_ref[...], w_ref[...]) + b_ref[...]
    h_ref[...] = h
    a_ref[...] = _dot(h, wa_ref[...])
    bt_ref[...] = _dot(h, wb_ref[...])


def _msg_body(pre_ref, ea_ref, wc_ref, b1_ref, w2_ref, b2_ref, out_ref):
    acc = pre_ref[...] + b1_ref[...]
    ea = ea_ref[...]
    for k in range(DE):
        acc = acc + ea[:, k:k + 1] * wc_ref[k:k + 1, :]
    m = jnp.maximum(acc, 0.0)
    m = jnp.maximum(_dot(m, w2_ref[...]) + b2_ref[...], 0.0)
    out_ref[...] = m


NAGG = NC * NHALF


def _sum_aggs(g_refs):
    s = g_refs[0][...]
    for g in g_refs[1:]:
        s = s + g[...]
    return s


def _update_body(h_ref, *rest):
    g_refs = rest[:NAGG]
    (u1h_ref, u1a_ref, u1b_ref, u2w_ref, u2b_ref, wa_ref, wb_ref,
     h_out, a_out, b_out) = rest[NAGG:]
    hv = h_ref[...]
    agg = _sum_aggs(g_refs)
    u = jnp.maximum(_dot(hv, u1h_ref[...]) + _dot(agg, u1a_ref[...])
                    + u1b_ref[...], 0.0)
    u = _dot(u, u2w_ref[...]) + u2b_ref[...]
    hn = hv + u
    h_out[...] = hn
    a_out[...] = _dot(hn, wa_ref[...])
    b_out[...] = _dot(hn, wb_ref[...])


def _final_body(h_ref, *rest):
    g_refs = rest[:NAGG]
    (u1h_ref, u1a_ref, u1b_ref, u2w_ref, u2b_ref, pp1_ref, pp1b_ref,
     pp2_ref, pp2b_ref, po1_ref, po1b_ref, po2_ref, po2b_ref,
     out_ref) = rest[NAGG:]
    hv = h_ref[...]
    agg = _sum_aggs(g_refs)
    u = jnp.maximum(_dot(hv, u1h_ref[...]) + _dot(agg, u1a_ref[...])
                    + u1b_ref[...], 0.0)
    u = _dot(u, u2w_ref[...]) + u2b_ref[...]
    hn = hv + u
    pp = jnp.maximum(_dot(hn, pp1_ref[...]) + pp1b_ref[...], 0.0)
    pp = _dot(pp, pp2_ref[...]) + pp2b_ref[...]
    q = jnp.maximum(_dot(pp, po1_ref[...]) + po1b_ref[...], 0.0)
    out_ref[...] = _dot(q, po2_ref[...]) + po2b_ref[...]


def _node_spec():
    return pl.BlockSpec((NB, H), lambda i: (i, 0))


def _full_spec(shape):
    return pl.BlockSpec(shape, lambda i: tuple(0 for _ in shape))


def _edge_spec(cols):
    return pl.BlockSpec((MB, cols), lambda i: (i, 0))


_embed_call = pl.pallas_call(
    _embed_body,
    grid=(NPAD // NB,),
    in_specs=[_node_spec(), _full_spec((H, H)), _full_spec((1, H)),
              _full_spec((H, H)), _full_spec((H, H))],
    out_specs=[_node_spec(), _node_spec(), _node_spec()],
    out_shape=[jax.ShapeDtypeStruct((NPAD, H), F32)] * 3,
)

_msg_call = pl.pallas_call(
    _msg_body,
    grid=(EH // MB,),
    in_specs=[_edge_spec(H), _edge_spec(DE),
              _full_spec((DE, H)), _full_spec((1, H)), _full_spec((H, H)),
              _full_spec((1, H))],
    out_specs=_edge_spec(H),
    out_shape=jax.ShapeDtypeStruct((EH, H), F32),
)

_update_call = pl.pallas_call(
    _update_body,
    grid=(NPAD // NB,),
    in_specs=[_node_spec()] * (1 + NAGG)
             + [_full_spec((H, H)), _full_spec((H, H)), _full_spec((1, H)),
                _full_spec((H, H)), _full_spec((1, H)),
                _full_spec((H, H)), _full_spec((H, H))],
    out_specs=[_node_spec(), _node_spec(), _node_spec()],
    out_shape=[jax.ShapeDtypeStruct((NPAD, H), F32)] * 3,
)

_final_call = pl.pallas_call(
    _final_body,
    grid=(NPAD // NB,),
    in_specs=[_node_spec()] * (1 + NAGG)
             + [_full_spec((H, H)), _full_spec((H, H)), _full_spec((1, H)),
                _full_spec((H, H)), _full_spec((1, H)),
                _full_spec((H, H)), _full_spec((1, H)),
                _full_spec((H, H)), _full_spec((1, H)),
                _full_spec((H, H)), _full_spec((1, H)),
                _full_spec((H, H)), _full_spec((1, H))],
    out_specs=_node_spec(),
    out_shape=jax.ShapeDtypeStruct((NPAD, H), F32),
)


def _row(v):
    return v.reshape(1, H)


def kernel(x, edge_index, edge_attr, params):
    ei = edge_index.astype(jnp.int32)
    src, dst = ei[0], ei[1]
    pad = E_PAD - E
    dst_p = jnp.concatenate([dst, jnp.zeros((pad,), jnp.int32)]) \
        .reshape(NW, NHALF, EPWH)
    src_p = jnp.concatenate([src, jnp.zeros((pad,), jnp.int32)]) \
        .reshape(NW, NHALF, EPWH)
    dst_s = jnp.concatenate([dst, jnp.full((pad,), DUMP, jnp.int32)]) \
        .reshape(NW, NHALF, EPWH)
    ea_p = jnp.concatenate([edge_attr.astype(F32),
                            jnp.zeros((pad, DE), F32)], axis=0) \
        .reshape(NW, NHALF, EPWH, DE)
    zpad = jnp.zeros((MAXG, GCHUNK), jnp.int32)
    idx_a = [jnp.concatenate(
        [dst_p[:, hf].reshape(EH // GCHUNK, GCHUNK), zpad])
        for hf in range(NHALF)]
    idx_b = [jnp.concatenate(
        [src_p[:, hf].reshape(EH // GCHUNK, GCHUNK), zpad])
        for hf in range(NHALF)]
    idx_s = [dst_s[:, hf].reshape(NW, SNCH, CHUNK) for hf in range(NHALF)]
    ea_h = [ea_p[:, hf].reshape(EH, DE) for hf in range(NHALF)]
    x_p = jnp.concatenate([x.astype(F32), jnp.zeros((NPAD - N, H), F32)],
                          axis=0)
    zeros_npad = jnp.zeros((NPAD, H), F32)

    layers = params['layers']
    wa0 = layers[0]['m1W'][:H]
    wb0 = layers[0]['m1W'][H:2 * H]
    h, A, B = _embed_call(x_p, params['emb_W'], _row(params['emb_b']),
                          wa0, wb0)

    out = None
    for li in range(len(layers)):
        lp = layers[li]
        wc = lp['m1W'][2 * H:]
        aggs = []
        pres = [None] * NHALF
        msgs = [None] * NHALF
        for hf in range(NHALF):
            pres[hf] = _sc_gather(A, B, idx_a[hf], idx_b[hf])
        for hf in range(NHALF):
            msgs[hf] = _msg_call(pres[hf], ea_h[hf], wc, _row(lp['m1b']),
                                 lp['m2W'], _row(lp['m2b']))
        for hf in range(NHALF):
            ap = _sc_scatter(msgs[hf], idx_s[hf], zeros_npad)
            aggs.extend([ap[0], ap[1]])
        u1h = lp['u1W'][:H]
        u1a = lp['u1W'][H:]
        if li + 1 < len(layers):
            nxt = layers[li + 1]
            h, A, B = _update_call(h, *aggs,
                                   u1h, u1a, _row(lp['u1b']), lp['u2W'],
                                   _row(lp['u2b']),
                                   nxt['m1W'][:H], nxt['m1W'][H:2 * H])
        else:
            out = _final_call(h, *aggs,
                              u1h, u1a, _row(lp['u1b']), lp['u2W'],
                              _row(lp['u2b']),
                              params['pp1W'], _row(params['pp1b']),
                              params['pp2W'], _row(params['pp2b']),
                              params['po1W'], _row(params['po1b']),
                              params['po2W'], _row(params['po2b']))
    return out[:N]
